# baseline (speedup 1.0000x reference)
import jax
import jax.numpy as jnp
import numpy as np
from jax.experimental import pallas as pl
from jax.experimental.pallas import tpu as pltpu


def _tc_body(ann_ref, logits_ref, masks_ref, out_ref, acc_ref):
    i = pl.program_id(0)
    j = pl.program_id(1)
    n_j = pl.num_programs(1)
    n_i = pl.num_programs(0)

    @pl.when(jnp.logical_and(i == 0, j == 0))
    def _init_out():
        out_ref[0, 0] = 0.0

    @pl.when(j == 0)
    def _init_acc():
        acc_ref[0, 0] = 0.0

    lg = logits_ref[0, 0]
    masks = masks_ref[0, 0]
    covered = jnp.sum(masks, axis=0) > 0.0
    outside = jnp.where(covered, 0.0, lg)
    acc_ref[0, 0] += jnp.sum(outside)

    @pl.when(j == n_j - 1)
    def _finish_pair():
        o = acc_ref[0, 0]
        b = i // 3
        c = i % 3
        err = jnp.where(o >= 0.0, o * o, 0.0) * ann_ref[b, c + 1]
        out_ref[0, 0] += err


def kernel(logits, box_masks, annotation_mask):
    B, C, H, W = logits.shape
    N = box_masks.shape[2]
    Hb = 128
    n_pairs = B * (C - 1)

    grid = (n_pairs, H // Hb)

    out = pl.pallas_call(
        _tc_body,
        grid=grid,
        in_specs=[
            pl.BlockSpec(memory_space=pltpu.SMEM),
            pl.BlockSpec(
                (1, 1, Hb, W),
                lambda i, j: (i // (C - 1), 1 + i % (C - 1), j, 0),
            ),
            pl.BlockSpec(
                (1, 1, N, Hb, W),
                lambda i, j: (i // (C - 1), 1 + i % (C - 1), 0, j, 0),
            ),
        ],
        out_specs=pl.BlockSpec(memory_space=pltpu.SMEM),
        out_shape=jax.ShapeDtypeStruct((1, 1), jnp.float32),
        scratch_shapes=[pltpu.SMEM((1, 1), jnp.float32)],
    )(annotation_mask, logits, box_masks)

    im_size = float(np.prod(logits.shape[2:]))
    return out[0, 0] / im_size


# speedup vs baseline: 1.3710x; 1.3710x over previous
import jax
import jax.numpy as jnp
import numpy as np
from jax.experimental import pallas as pl
from jax.experimental.pallas import tpu as pltpu


def _tc_body(ann_ref, logits_ref, masks_ref, out_ref, acc_ref):
    i = pl.program_id(0)
    j = pl.program_id(1)
    n_j = pl.num_programs(1)
    n_i = pl.num_programs(0)

    @pl.when(jnp.logical_and(i == 0, j == 0))
    def _init_out():
        out_ref[0, 0] = 0.0

    @pl.when(j == 0)
    def _init_acc():
        acc_ref[0, 0] = 0.0

    lg = logits_ref[0, 0]
    masks = masks_ref[0, 0]
    covered = jnp.sum(masks, axis=0) > 0.0
    outside = jnp.where(covered, 0.0, lg)
    acc_ref[0, 0] += jnp.sum(outside)

    @pl.when(j == n_j - 1)
    def _finish_pair():
        o = acc_ref[0, 0]
        b = i // 3
        c = i % 3
        err = jnp.where(o >= 0.0, o * o, 0.0) * ann_ref[b, c + 1]
        out_ref[0, 0] += err


def kernel(logits, box_masks, annotation_mask):
    B, C, H, W = logits.shape
    N = box_masks.shape[2]
    Hb = 256
    n_pairs = B * (C - 1)

    grid = (n_pairs, H // Hb)

    out = pl.pallas_call(
        _tc_body,
        grid=grid,
        in_specs=[
            pl.BlockSpec(memory_space=pltpu.SMEM),
            pl.BlockSpec(
                (1, 1, Hb, W),
                lambda i, j: (i // (C - 1), 1 + i % (C - 1), j, 0),
            ),
            pl.BlockSpec(
                (1, 1, N, Hb, W),
                lambda i, j: (i // (C - 1), 1 + i % (C - 1), 0, j, 0),
            ),
        ],
        out_specs=pl.BlockSpec(memory_space=pltpu.SMEM),
        out_shape=jax.ShapeDtypeStruct((1, 1), jnp.float32),
        scratch_shapes=[pltpu.SMEM((1, 1), jnp.float32)],
    )(annotation_mask, logits, box_masks)

    im_size = float(np.prod(logits.shape[2:]))
    return out[0, 0] / im_size


# speedup vs baseline: 1.4480x; 1.0562x over previous
import jax
import jax.numpy as jnp
import numpy as np
from jax.experimental import pallas as pl
from jax.experimental.pallas import tpu as pltpu


def _tc_body(ann_ref, logits_ref, masks_ref, out_ref, acc_ref):
    i = pl.program_id(0)
    j = pl.program_id(1)
    n_j = pl.num_programs(1)
    n_i = pl.num_programs(0)

    @pl.when(jnp.logical_and(i == 0, j == 0))
    def _init_out():
        out_ref[0, 0] = 0.0

    @pl.when(j == 0)
    def _init_acc():
        acc_ref[0, 0] = 0.0

    lg = logits_ref[0, 0]
    masks = masks_ref[0, 0]
    covered = jnp.sum(masks, axis=0) > 0.0
    outside = jnp.where(covered, 0.0, lg)
    acc_ref[0, 0] += jnp.sum(outside)

    @pl.when(j == n_j - 1)
    def _finish_pair():
        o = acc_ref[0, 0]
        b = i // 3
        c = i % 3
        err = jnp.where(o >= 0.0, o * o, 0.0) * ann_ref[b, c + 1]
        out_ref[0, 0] += err


def kernel(logits, box_masks, annotation_mask):
    B, C, H, W = logits.shape
    N = box_masks.shape[2]
    Hb = 512
    n_pairs = B * (C - 1)

    grid = (n_pairs, H // Hb)

    out = pl.pallas_call(
        _tc_body,
        grid=grid,
        in_specs=[
            pl.BlockSpec(memory_space=pltpu.SMEM),
            pl.BlockSpec(
                (1, 1, Hb, W),
                lambda i, j: (i // (C - 1), 1 + i % (C - 1), j, 0),
            ),
            pl.BlockSpec(
                (1, 1, N, Hb, W),
                lambda i, j: (i // (C - 1), 1 + i % (C - 1), 0, j, 0),
            ),
        ],
        out_specs=pl.BlockSpec(memory_space=pltpu.SMEM),
        out_shape=jax.ShapeDtypeStruct((1, 1), jnp.float32),
        scratch_shapes=[pltpu.SMEM((1, 1), jnp.float32)],
    )(annotation_mask, logits, box_masks)

    im_size = float(np.prod(logits.shape[2:]))
    return out[0, 0] / im_size
